# baseline (speedup 1.0000x reference)
import functools

import jax
import jax.numpy as jnp
from jax import lax
from jax.experimental import pallas as pl
from jax.experimental.pallas import tpu as pltpu
from jax.experimental.pallas import tpu_sc as plsc

D_MODEL = 1024
DBLK = D_MODEL // 128
BATCH = 4
SEQ_LEN = 2048
OUT_LEN = SEQ_LEN + 1
SEQ_VOCAB = 30
NUM_WORKERS = 32
NREP = 32
L_PER_WORKER = SEQ_LEN // NUM_WORKERS
ROWBLK = BATCH * DBLK
CHUNK_L = 8
NCHUNKS = L_PER_WORKER // CHUNK_L
PIECES = CHUNK_L * ROWBLK
LANES = 16


def _encode(tok3, intensity_ids, seq_pieces, int_pieces):
    mesh = plsc.VectorSubcoreMesh(core_axis_name="c", subcore_axis_name="s")

    @functools.partial(
        pl.kernel,
        mesh=mesh,
        out_type=jax.ShapeDtypeStruct((OUT_LEN * ROWBLK, 128), jnp.float32),
        scratch_types=[
            pltpu.VMEM((8, 128), jnp.int32),
            pltpu.VMEM((2, 2, 128), jnp.int32),
            pltpu.VMEM((2, PIECES, 128), jnp.float32),
            pltpu.VMEM((LANES,), jnp.int32),
            pltpu.VMEM((2 * LANES,), jnp.int32),
            pltpu.VMEM((2 * LANES, 128), jnp.float32),
            pltpu.SemaphoreType.DMA,
            pltpu.SemaphoreType.DMA,
            pltpu.SemaphoreType.DMA,
        ],
    )
    def k(tok_hbm, iid_hbm, seqp_hbm, intp_hbm, out_hbm,
          tokv, pidx, buf, iidx_v, ipidx_v, ibuf, sem0, sem1, isem):
        wid = lax.axis_index("s") * 2 + lax.axis_index("c")
        pltpu.sync_copy(tok_hbm.at[pl.ds((wid // 4) * 8, 8)], tokv)
        rowbase = (wid % 4) * 2

        lane = lax.iota(jnp.int32, LANES)
        bvec = lane & (BATCH - 1)
        woff = wid * (SEQ_VOCAB * DBLK)
        sems = (sem0, sem1)

        def _perm(window, start):
            return lax.gather(
                window, (start + bvec)[:, None],
                dimension_numbers=lax.GatherDimensionNumbers(
                    offset_dims=(), collapsed_slice_dims=(0,),
                    start_index_map=(0,)),
                slice_sizes=(1,),
                mode=lax.GatherScatterMode.PROMISE_IN_BOUNDS)

        def build_idx(jc):
            par = jc % 2
            row = rowbase + jc // 4
            w0 = tokv[row, pl.ds((jc % 4) * 32, LANES)]
            w1 = tokv[row, pl.ds((jc % 4) * 32 + LANES, LANES)]
            for v in range(LANES):
                kvec = ((lane >> 2) + 4 * v) & (DBLK - 1)
                foff = (v // 2) * 4
                tokval = _perm(w0 if foff < LANES else w1, foff % LANES)
                g = tokval * DBLK + (kvec + woff)
                pidx[par, v // 8, pl.ds((v % 8) * LANES, LANES)] = g

        def fire(jc):
            par = jc % 2
            return (
                pltpu.async_copy(seqp_hbm.at[pidx.at[par, 0]],
                                 buf.at[par, pl.ds(0, 128)], sems[par]),
                pltpu.async_copy(seqp_hbm.at[pidx.at[par, 1]],
                                 buf.at[par, pl.ds(128, 128)], sems[par]),
            )

        build_idx(0)
        copies = [None] * NCHUNKS
        copies[0] = fire(0)
        for jc in range(NCHUNKS):
            if jc + 1 < NCHUNKS:
                build_idx(jc + 1)
                copies[jc + 1] = fire(jc + 1)
            copies[jc][0].wait()
            copies[jc][1].wait()
            dst = (wid * L_PER_WORKER + jc * CHUNK_L) * ROWBLK
            pltpu.sync_copy(buf.at[jc % 2], out_hbm.at[pl.ds(dst, PIECES)])

        @pl.when(wid == 0)
        def _():
            pltpu.sync_copy(iid_hbm, iidx_v.at[pl.ds(0, BATCH)])
            ids16 = iidx_v[pl.ds(0, LANES)]
            for v in range(2):
                kvec = ((lane >> 2) + 4 * v) & (DBLK - 1)
                rid = _perm(ids16, 0)
                pieces = (rid >> 3) * 64 + kvec * DBLK + (rid & 7)
                ipidx_v[pl.ds(v * LANES, LANES)] = pieces
            pltpu.async_copy(intp_hbm.at[ipidx_v], ibuf, isem).wait()
            pltpu.sync_copy(ibuf,
                            out_hbm.at[pl.ds(SEQ_LEN * ROWBLK, ROWBLK)])

    return k(tok3, intensity_ids, seq_pieces, int_pieces)


def kernel(sequence_tokens, intensity_ids, seq_table, int_table):
    tok3 = (sequence_tokens.astype(jnp.int32)
            .transpose(1, 0).reshape(SEQ_LEN // 32, 128))
    int_pieces = (int_table.reshape(8, DBLK, DBLK, 128)
                  .transpose(0, 2, 1, 3).reshape(64 * DBLK, 128))
    seq_pieces = jnp.tile(seq_table, (NREP, 1)).reshape(
        NREP * SEQ_VOCAB * DBLK, 128)
    out2 = _encode(tok3, intensity_ids.astype(jnp.int32), seq_pieces,
                   int_pieces)
    return (out2.reshape(OUT_LEN, DBLK, BATCH, 128)
            .transpose(2, 0, 1, 3)
            .reshape(BATCH, OUT_LEN, D_MODEL))


# speedup vs baseline: 1.0722x; 1.0722x over previous
import functools

import jax
import jax.numpy as jnp
from jax import lax
from jax.experimental import pallas as pl
from jax.experimental.pallas import tpu as pltpu
from jax.experimental.pallas import tpu_sc as plsc

D_MODEL = 1024
DBLK = D_MODEL // 128
BATCH = 4
SEQ_LEN = 2048
OUT_LEN = SEQ_LEN + 1
SEQ_VOCAB = 30
NUM_WORKERS = 32
NREP = 32
ROWS_PER_WORKER = BATCH * SEQ_LEN // NUM_WORKERS
WORKERS_PER_BATCH = NUM_WORKERS // BATCH
TOKBLK = ROWS_PER_WORKER // 128
CHUNK = 32
NCHUNKS = ROWS_PER_WORKER // CHUNK
LANES = 16


def _encode(tok3, intensity_ids, seq_rep, int_pieces):
    mesh = plsc.VectorSubcoreMesh(core_axis_name="c", subcore_axis_name="s")

    @functools.partial(
        pl.kernel,
        mesh=mesh,
        out_type=jax.ShapeDtypeStruct((OUT_LEN, DBLK, BATCH, 128),
                                      jnp.float32),
        scratch_types=[
            pltpu.VMEM((TOKBLK, 128), jnp.int32),
            pltpu.VMEM((CHUNK, DBLK, 128), jnp.float32),
            pltpu.VMEM((CHUNK, DBLK, 128), jnp.float32),
            pltpu.VMEM((CHUNK, DBLK, 128), jnp.float32),
            pltpu.VMEM((LANES,), jnp.int32),
            pltpu.VMEM((2 * LANES,), jnp.int32),
            pltpu.VMEM((2 * LANES, 128), jnp.float32),
            pltpu.SemaphoreType.DMA,
            pltpu.SemaphoreType.DMA,
            pltpu.SemaphoreType.DMA,
            pltpu.SemaphoreType.DMA,
            pltpu.SemaphoreType.DMA,
            pltpu.SemaphoreType.DMA,
            pltpu.SemaphoreType.DMA,
        ],
    )
    def k(tok_hbm, iid_hbm, seqt_hbm, intt_hbm, out_hbm,
          idx_v, buf0, buf1, buf2, iidx_v, ipidx_v, ibuf,
          sem0, sem1, sem2, wsem0, wsem1, wsem2, isem):
        sid = lax.axis_index("s")
        wid = sid * 2 + lax.axis_index("c")
        b = wid // WORKERS_PER_BATCH
        part = wid % WORKERS_PER_BATCH
        dst_base = part * ROWS_PER_WORKER

        pltpu.sync_copy(tok_hbm.at[pl.ds(part * TOKBLK, TOKBLK), b, :],
                        idx_v)

        off = jnp.full((LANES,), SEQ_VOCAB, jnp.int32) * wid
        for r in range(TOKBLK):
            for c in range(128 // LANES):
                sl = pl.ds(c * LANES, LANES)
                idx_v[r, sl] = idx_v[r, sl] + off

        bufs = (buf0, buf1, buf2)
        sems = (sem0, sem1, sem2)
        wsems = (wsem0, wsem1, wsem2)

        def chunk_idx(j):
            return idx_v.at[j // 4, pl.ds((j % 4) * CHUNK, CHUNK)]

        def fire_gather(j):
            return pltpu.async_copy(seqt_hbm.at[chunk_idx(j)],
                                    bufs[j % 3], sems[j % 3])

        copies = [None] * NCHUNKS
        writes = [None] * NCHUNKS
        copies[0] = fire_gather(0)
        copies[1] = fire_gather(1)
        for j in range(NCHUNKS):
            if j + 2 < NCHUNKS:
                if j - 1 >= 0:
                    writes[j - 1].wait()
                copies[j + 2] = fire_gather(j + 2)
            copies[j].wait()
            writes[j] = pltpu.async_copy(
                bufs[j % 3],
                out_hbm.at[pl.ds(dst_base + j * CHUNK, CHUNK), :, b, :],
                wsems[j % 3])
        writes[NCHUNKS - 3].wait()
        writes[NCHUNKS - 2].wait()
        writes[NCHUNKS - 1].wait()

        @pl.when(wid == 0)
        def _():
            pltpu.sync_copy(iid_hbm, iidx_v.at[pl.ds(0, BATCH)])
            ids16 = iidx_v[pl.ds(0, LANES)]
            for v in range(2):
                j = lax.iota(jnp.int32, LANES) + (v * LANES)
                sel = j >> 3
                iv = j & (DBLK - 1)
                rid = lax.gather(
                    ids16, sel[:, None],
                    dimension_numbers=lax.GatherDimensionNumbers(
                        offset_dims=(), collapsed_slice_dims=(0,),
                        start_index_map=(0,)),
                    slice_sizes=(1,),
                    mode=lax.GatherScatterMode.PROMISE_IN_BOUNDS)
                pieces = (rid >> 3) * 64 + iv * 8 + (rid & 7)
                ipidx_v[pl.ds(v * LANES, LANES)] = pieces
            pltpu.async_copy(intt_hbm.at[ipidx_v], ibuf, isem).wait()
            for bb in range(BATCH):
                pltpu.sync_copy(ibuf.at[pl.ds(bb * DBLK, DBLK)],
                                out_hbm.at[SEQ_LEN, :, bb, :])

    return k(tok3, intensity_ids, seq_rep, int_pieces)


def kernel(sequence_tokens, intensity_ids, seq_table, int_table):
    tok3 = (sequence_tokens.astype(jnp.int32)
            .reshape(BATCH, SEQ_LEN // 128, 128).transpose(1, 0, 2))
    int_pieces = (int_table.reshape(8, DBLK, DBLK, 128)
                  .transpose(0, 2, 1, 3).reshape(64 * DBLK, 128))
    seq_rep = jnp.tile(seq_table, (NREP, 1)).reshape(
        NREP * SEQ_VOCAB, DBLK, 128)
    out4 = _encode(tok3, intensity_ids.astype(jnp.int32), seq_rep,
                   int_pieces)
    return out4.transpose(2, 0, 1, 3).reshape(BATCH, OUT_LEN, D_MODEL)


# speedup vs baseline: 1.0770x; 1.0045x over previous
import functools

import jax
import jax.numpy as jnp
from jax import lax
from jax.experimental import pallas as pl
from jax.experimental.pallas import tpu as pltpu
from jax.experimental.pallas import tpu_sc as plsc

D_MODEL = 1024
DBLK = D_MODEL // 128
BATCH = 4
SEQ_LEN = 2048
OUT_LEN = SEQ_LEN + 1
SEQ_VOCAB = 30
NUM_WORKERS = 32
NREP = 32
ROWS_PER_WORKER = BATCH * SEQ_LEN // NUM_WORKERS
WORKERS_PER_BATCH = NUM_WORKERS // BATCH
TOKBLK = ROWS_PER_WORKER // 128
CHUNK = 32
NCHUNKS = ROWS_PER_WORKER // CHUNK
LANES = 16


def _encode(tok3, intensity_ids, seq_rep, int_pieces):
    mesh = plsc.VectorSubcoreMesh(core_axis_name="c", subcore_axis_name="s")

    @functools.partial(
        pl.kernel,
        mesh=mesh,
        out_type=jax.ShapeDtypeStruct((OUT_LEN, DBLK, BATCH, 128),
                                      jnp.float32),
        scratch_types=[
            pltpu.VMEM((TOKBLK, 128), jnp.int32),
            pltpu.VMEM((CHUNK, DBLK, 128), jnp.float32),
            pltpu.VMEM((CHUNK, DBLK, 128), jnp.float32),
            pltpu.VMEM((CHUNK, DBLK, 128), jnp.float32),
            pltpu.VMEM((LANES,), jnp.int32),
            pltpu.VMEM((2 * LANES,), jnp.int32),
            pltpu.VMEM((2 * LANES, 128), jnp.float32),
            pltpu.SemaphoreType.DMA,
            pltpu.SemaphoreType.DMA,
            pltpu.SemaphoreType.DMA,
            pltpu.SemaphoreType.DMA,
            pltpu.SemaphoreType.DMA,
            pltpu.SemaphoreType.DMA,
            pltpu.SemaphoreType.DMA,
        ],
    )
    def k(tok_hbm, iid_hbm, seqt_hbm, intt_hbm, out_hbm,
          idx_v, buf0, buf1, buf2, iidx_v, ipidx_v, ibuf,
          sem0, sem1, sem2, wsem0, wsem1, wsem2, isem):
        sid = lax.axis_index("s")
        wid = sid * 2 + lax.axis_index("c")
        b = wid // WORKERS_PER_BATCH
        part = wid % WORKERS_PER_BATCH
        dst_base = part * ROWS_PER_WORKER

        pltpu.sync_copy(tok_hbm.at[pl.ds(part * TOKBLK, TOKBLK), b, :],
                        idx_v)

        off = jnp.full((LANES,), SEQ_VOCAB, jnp.int32) * wid
        for r in range(TOKBLK):
            for c in range(128 // LANES):
                sl = pl.ds(c * LANES, LANES)
                idx_v[r, sl] = idx_v[r, sl] + off

        bufs = (buf0, buf1, buf2)
        sems = (sem0, sem1, sem2)
        wsems = (wsem0, wsem1, wsem2)

        def chunk_idx(j):
            return idx_v.at[j // 4, pl.ds((j % 4) * CHUNK, CHUNK)]

        def fire_gather(j):
            return pltpu.async_copy(seqt_hbm.at[chunk_idx(j)],
                                    bufs[j % 3], sems[j % 3])

        copies = [None] * NCHUNKS
        writes = [None] * NCHUNKS
        copies[0] = fire_gather(0)
        copies[1] = fire_gather(1)
        for j in range(NCHUNKS):
            if j + 2 < NCHUNKS:
                if j - 1 >= 0:
                    writes[j - 1].wait()
                copies[j + 2] = fire_gather(j + 2)
            copies[j].wait()
            writes[j] = pltpu.async_copy(
                bufs[j % 3],
                out_hbm.at[pl.ds(dst_base + j * CHUNK, CHUNK), :, b, :],
                wsems[j % 3])
        writes[NCHUNKS - 3].wait()
        writes[NCHUNKS - 2].wait()
        writes[NCHUNKS - 1].wait()

        @pl.when(wid < BATCH)
        def _():
            pltpu.sync_copy(iid_hbm, iidx_v.at[pl.ds(0, BATCH)])
            ids16 = iidx_v[pl.ds(0, LANES)]
            widvec = jnp.full((LANES,), 1, jnp.int32) * wid
            rid = lax.gather(
                ids16, widvec[:, None],
                dimension_numbers=lax.GatherDimensionNumbers(
                    offset_dims=(), collapsed_slice_dims=(0,),
                    start_index_map=(0,)),
                slice_sizes=(1,),
                mode=lax.GatherScatterMode.PROMISE_IN_BOUNDS)
            iv = lax.iota(jnp.int32, LANES) & (DBLK - 1)
            ipidx_v[pl.ds(0, LANES)] = (rid >> 3) * 64 + iv * 8 + (rid & 7)
            pltpu.async_copy(intt_hbm.at[ipidx_v.at[pl.ds(0, LANES)]],
                             ibuf.at[pl.ds(0, LANES)], isem).wait()
            pltpu.sync_copy(ibuf.at[pl.ds(0, DBLK)],
                            out_hbm.at[SEQ_LEN, :, wid, :])

    return k(tok3, intensity_ids, seq_rep, int_pieces)


def kernel(sequence_tokens, intensity_ids, seq_table, int_table):
    tok3 = (sequence_tokens.astype(jnp.int32)
            .reshape(BATCH, SEQ_LEN // 128, 128).transpose(1, 0, 2))
    int_pieces = (int_table.reshape(8, DBLK, DBLK, 128)
                  .transpose(0, 2, 1, 3).reshape(64 * DBLK, 128))
    seq_rep = jnp.tile(seq_table, (NREP, 1)).reshape(
        NREP * SEQ_VOCAB, DBLK, 128)
    out4 = _encode(tok3, intensity_ids.astype(jnp.int32), seq_rep,
                   int_pieces)
    return out4.transpose(2, 0, 1, 3).reshape(BATCH, OUT_LEN, D_MODEL)
